# baseline (speedup 1.0000x reference)
import functools

import jax
import jax.numpy as jnp
from jax.experimental import pallas as pl
from jax.experimental.pallas import tpu as pltpu

_B, _T, _C = 16, 32768, 64
_ROWS = _B * _T
_BLK_ROWS = 4096


def _loss_body(p_ref, t_ref, m_ref, se_ref, n_ref):
    i = pl.program_id(0)

    @pl.when(i == 0)
    def _():
        se_ref[...] = jnp.zeros_like(se_ref)
        n_ref[...] = jnp.zeros_like(n_ref)

    d = p_ref[...] - t_ref[...]
    m = m_ref[...]
    se_ref[...] += jnp.sum(d * d * m, keepdims=True)
    n_ref[...] += jnp.sum(m, keepdims=True)


def kernel(predicted, target, mask):
    p2 = predicted.reshape(_ROWS, _C)
    t2 = target.reshape(_ROWS, _C)
    mf = mask.reshape(_ROWS, 1).astype(jnp.float32)

    grid = (_ROWS // _BLK_ROWS,)
    se_sum, n_sum = pl.pallas_call(
        _loss_body,
        grid=grid,
        in_specs=[
            pl.BlockSpec((_BLK_ROWS, _C), lambda i: (i, 0)),
            pl.BlockSpec((_BLK_ROWS, _C), lambda i: (i, 0)),
            pl.BlockSpec((_BLK_ROWS, 1), lambda i: (i, 0)),
        ],
        out_specs=[
            pl.BlockSpec((1, 1), lambda i: (0, 0)),
            pl.BlockSpec((1, 1), lambda i: (0, 0)),
        ],
        out_shape=[
            jax.ShapeDtypeStruct((1, 1), jnp.float32),
            jax.ShapeDtypeStruct((1, 1), jnp.float32),
        ],
        compiler_params=pltpu.CompilerParams(
            dimension_semantics=("arbitrary",),
        ),
    )(p2, t2, mf)

    se = se_sum[0, 0]
    n = n_sum[0, 0]
    count = n * jnp.float32(_C)
    safe = jnp.where(count == 0.0, jnp.float32(1.0), count)
    return jnp.where(n == 0.0, jnp.float32(0.0), se / safe)
